# initial kernel scaffold (unmeasured)
import jax
import jax.numpy as jnp
from jax import lax
from jax.experimental import pallas as pl
from jax.experimental.pallas import tpu as pltpu

N_DEV = 16
B, SQ, D = 4, 256, 1024
H_LOC, DH = 8, 128
ROWS = B * SQ
CHUNK = ROWS // N_DEV
SCALE = 0.08838834764831843


def kernel(x, Wq, Wo, Wk, Wv):
    xb = x.reshape(ROWS, D).astype(jnp.bfloat16)
    wq = Wq.astype(jnp.bfloat16)
    wk = Wk.astype(jnp.bfloat16)
    wv = Wv.astype(jnp.bfloat16)
    wo = Wo.astype(jnp.bfloat16)

    def body(x_ref, wq_ref, wk_ref, wv_ref, wo_ref, out_ref,
             acc_ref, attn_ref, rs_ref,
             rs_send, rs_recv, ag_send, ag_recv):
        d = lax.axis_index("i")
        right = lax.rem(d + 1, N_DEV)

        xv = x_ref[:]
        q = jnp.dot(xv, wq_ref[:], preferred_element_type=jnp.bfloat16)
        k = jnp.dot(xv, wk_ref[:], preferred_element_type=jnp.bfloat16)
        v = jnp.dot(xv, wv_ref[:], preferred_element_type=jnp.bfloat16)

        for b in range(B):
            for h in range(H_LOC):
                qs = q[b * SQ:(b + 1) * SQ, h * DH:(h + 1) * DH]
                ks = k[b * SQ:(b + 1) * SQ, h * DH:(h + 1) * DH]
                vs = v[b * SQ:(b + 1) * SQ, h * DH:(h + 1) * DH]
                s = lax.dot_general(
                    qs, ks, (((1,), (1,)), ((), ())),
                    preferred_element_type=jnp.float32,
                ) * SCALE
                m = jnp.max(s, axis=1, keepdims=True)
                p = jnp.exp(s - m)
                l = jnp.sum(p, axis=1, keepdims=True)
                o = jnp.dot(p.astype(jnp.bfloat16), vs,
                            preferred_element_type=jnp.float32) / l
                attn_ref[b * SQ:(b + 1) * SQ, h * DH:(h + 1) * DH] = (
                    o.astype(jnp.bfloat16))

        acc_ref[:] = jnp.dot(attn_ref[:], wo_ref[:],
                             preferred_element_type=jnp.float32)

        for st in range(N_DEV - 1):
            c = lax.rem(d - st + N_DEV, N_DEV)
            rdma = pltpu.make_async_remote_copy(
                src_ref=acc_ref.at[pl.ds(c * CHUNK, CHUNK), :],
                dst_ref=rs_ref.at[st],
                send_sem=rs_send.at[st],
                recv_sem=rs_recv.at[st],
                device_id=(right,),
                device_id_type=pl.DeviceIdType.MESH,
            )
            rdma.start()
            rdma.wait()
            cr = lax.rem(d - 1 - st + 2 * N_DEV, N_DEV)
            acc_ref[pl.ds(cr * CHUNK, CHUNK), :] = (
                acc_ref[pl.ds(cr * CHUNK, CHUNK), :] + rs_ref[st])

        cm = lax.rem(d + 1, N_DEV)
        out_ref[pl.ds(cm * CHUNK, CHUNK), :] = (
            acc_ref[pl.ds(cm * CHUNK, CHUNK), :])

        for st in range(N_DEV - 1):
            c = lax.rem(d + 1 - st + 2 * N_DEV, N_DEV)
            rdma = pltpu.make_async_remote_copy(
                src_ref=out_ref.at[pl.ds(c * CHUNK, CHUNK), :],
                dst_ref=out_ref.at[pl.ds(c * CHUNK, CHUNK), :],
                send_sem=ag_send.at[st],
                recv_sem=ag_recv.at[st],
                device_id=(right,),
                device_id_type=pl.DeviceIdType.MESH,
            )
            rdma.start()
            rdma.wait()

    out2 = pl.pallas_call(
        body,
        out_shape=jax.ShapeDtypeStruct((ROWS, D), jnp.float32),
        in_specs=[pl.BlockSpec(memory_space=pltpu.VMEM)] * 5,
        out_specs=pl.BlockSpec(memory_space=pltpu.VMEM),
        scratch_shapes=[
            pltpu.VMEM((ROWS, D), jnp.float32),
            pltpu.VMEM((ROWS, D), jnp.bfloat16),
            pltpu.VMEM((N_DEV - 1, CHUNK, D), jnp.float32),
            pltpu.SemaphoreType.DMA((N_DEV - 1,)),
            pltpu.SemaphoreType.DMA((N_DEV - 1,)),
            pltpu.SemaphoreType.DMA((N_DEV - 1,)),
            pltpu.SemaphoreType.DMA((N_DEV - 1,)),
        ],
    )(xb, wq, wk, wv, wo)
    return out2.reshape(B, SQ, D)


# baseline (device time: 173477 ns/iter reference)
import jax
import jax.numpy as jnp
from jax import lax
from jax.experimental import pallas as pl
from jax.experimental.pallas import tpu as pltpu

N_DEV = 16
B, SQ, D = 4, 256, 1024
H_LOC, DH = 8, 128
ROWS = B * SQ
CHUNK = ROWS // N_DEV
SCALE = 0.08838834764831843


def kernel(x, Wq, Wo, Wk, Wv):
    xb = x.reshape(ROWS, D).astype(jnp.bfloat16)
    wq = Wq.astype(jnp.bfloat16)
    wk = Wk.astype(jnp.bfloat16)
    wv = Wv.astype(jnp.bfloat16)
    wo = Wo.astype(jnp.bfloat16)

    def body(x_ref, wq_ref, wk_ref, wv_ref, wo_ref, out_ref,
             acc_ref, attn_ref, rs_ref,
             rs_send, rs_recv, ag_send, ag_recv):
        d = lax.axis_index("i")
        right = lax.rem(d + 1, N_DEV)

        xv = x_ref[:]
        q = jnp.dot(xv, wq_ref[:],
                    preferred_element_type=jnp.float32).astype(jnp.bfloat16)
        k = jnp.dot(xv, wk_ref[:],
                    preferred_element_type=jnp.float32).astype(jnp.bfloat16)
        v = jnp.dot(xv, wv_ref[:],
                    preferred_element_type=jnp.float32).astype(jnp.bfloat16)

        for b in range(B):
            for h in range(H_LOC):
                qs = q[b * SQ:(b + 1) * SQ, h * DH:(h + 1) * DH]
                ks = k[b * SQ:(b + 1) * SQ, h * DH:(h + 1) * DH]
                vs = v[b * SQ:(b + 1) * SQ, h * DH:(h + 1) * DH]
                s = lax.dot_general(
                    qs, ks, (((1,), (1,)), ((), ())),
                    preferred_element_type=jnp.float32,
                ) * SCALE
                m = jnp.max(s, axis=1, keepdims=True)
                p = jnp.exp(s - m)
                l = jnp.sum(p, axis=1, keepdims=True)
                o = jnp.dot(p.astype(jnp.bfloat16), vs,
                            preferred_element_type=jnp.float32) / l
                attn_ref[b * SQ:(b + 1) * SQ, h * DH:(h + 1) * DH] = (
                    o.astype(jnp.bfloat16))

        acc_ref[:] = jnp.dot(attn_ref[:], wo_ref[:],
                             preferred_element_type=jnp.float32)

        for st in range(N_DEV - 1):
            c = lax.rem(d - st + N_DEV, N_DEV)
            rdma = pltpu.make_async_remote_copy(
                src_ref=acc_ref.at[pl.ds(c * CHUNK, CHUNK), :],
                dst_ref=rs_ref.at[st],
                send_sem=rs_send.at[st],
                recv_sem=rs_recv.at[st],
                device_id=(right,),
                device_id_type=pl.DeviceIdType.MESH,
            )
            rdma.start()
            rdma.wait()
            cr = lax.rem(d - 1 - st + 2 * N_DEV, N_DEV)
            acc_ref[pl.ds(cr * CHUNK, CHUNK), :] = (
                acc_ref[pl.ds(cr * CHUNK, CHUNK), :] + rs_ref[st])

        cm = lax.rem(d + 1, N_DEV)
        out_ref[pl.ds(cm * CHUNK, CHUNK), :] = (
            acc_ref[pl.ds(cm * CHUNK, CHUNK), :])

        for st in range(N_DEV - 1):
            c = lax.rem(d + 1 - st + 2 * N_DEV, N_DEV)
            rdma = pltpu.make_async_remote_copy(
                src_ref=out_ref.at[pl.ds(c * CHUNK, CHUNK), :],
                dst_ref=out_ref.at[pl.ds(c * CHUNK, CHUNK), :],
                send_sem=ag_send.at[st],
                recv_sem=ag_recv.at[st],
                device_id=(right,),
                device_id_type=pl.DeviceIdType.MESH,
            )
            rdma.start()
            rdma.wait()

    out2 = pl.pallas_call(
        body,
        out_shape=jax.ShapeDtypeStruct((ROWS, D), jnp.float32),
        in_specs=[pl.BlockSpec(memory_space=pltpu.VMEM)] * 5,
        out_specs=pl.BlockSpec(memory_space=pltpu.VMEM),
        scratch_shapes=[
            pltpu.VMEM((ROWS, D), jnp.float32),
            pltpu.VMEM((ROWS, D), jnp.bfloat16),
            pltpu.VMEM((N_DEV - 1, CHUNK, D), jnp.float32),
            pltpu.SemaphoreType.DMA((N_DEV - 1,)),
            pltpu.SemaphoreType.DMA((N_DEV - 1,)),
            pltpu.SemaphoreType.DMA((N_DEV - 1,)),
            pltpu.SemaphoreType.DMA((N_DEV - 1,)),
        ],
    )(xb, wq, wk, wv, wo)
    return out2.reshape(B, SQ, D)


# device time: 131517 ns/iter; 1.3190x vs baseline; 1.3190x over previous
import jax
import jax.numpy as jnp
from jax import lax
from jax.experimental import pallas as pl
from jax.experimental.pallas import tpu as pltpu

N_DEV = 16
B, SQ, D = 4, 256, 1024
H_LOC, DH = 8, 128
ROWS = B * SQ
CHUNK = ROWS // N_DEV
SCALE = 0.08838834764831843


def kernel(x, Wq, Wo, Wk, Wv):
    xb = x.reshape(ROWS, D).astype(jnp.bfloat16)
    wq = Wq.astype(jnp.bfloat16)
    wk = Wk.astype(jnp.bfloat16)
    wv = Wv.astype(jnp.bfloat16)
    wo = Wo.astype(jnp.bfloat16)

    def body(x_ref, wq_ref, wk_ref, wv_ref, wo_ref, out_ref,
             acc_ref, attn_ref, rs_ref, stage_ref,
             rs_send, rs_recv, ag_send, ag_recv):
        d = lax.axis_index("i")
        right = lax.rem(d + 1, N_DEV)

        xv = x_ref[:]
        q = jnp.dot(xv, wq_ref[:],
                    preferred_element_type=jnp.float32).astype(jnp.bfloat16)
        k = jnp.dot(xv, wk_ref[:],
                    preferred_element_type=jnp.float32).astype(jnp.bfloat16)
        v = jnp.dot(xv, wv_ref[:],
                    preferred_element_type=jnp.float32).astype(jnp.bfloat16)

        for b in range(B):
            for h in range(H_LOC):
                qs = q[b * SQ:(b + 1) * SQ, h * DH:(h + 1) * DH]
                ks = k[b * SQ:(b + 1) * SQ, h * DH:(h + 1) * DH]
                vs = v[b * SQ:(b + 1) * SQ, h * DH:(h + 1) * DH]
                s = lax.dot_general(
                    qs, ks, (((1,), (1,)), ((), ())),
                    preferred_element_type=jnp.float32,
                ) * SCALE
                m = jnp.max(s, axis=1, keepdims=True)
                p = jnp.exp(s - m)
                l = jnp.sum(p, axis=1, keepdims=True)
                o = jnp.dot(p.astype(jnp.bfloat16), vs,
                            preferred_element_type=jnp.float32) / l
                attn_ref[b * SQ:(b + 1) * SQ, h * DH:(h + 1) * DH] = (
                    o.astype(jnp.bfloat16))

        acc_ref[:] = jnp.dot(attn_ref[:], wo_ref[:],
                             preferred_element_type=jnp.float32)

        for st in range(N_DEV - 1):
            c = lax.rem(d - st + N_DEV, N_DEV)
            stage_ref[:] = acc_ref[pl.ds(c * CHUNK, CHUNK), :].astype(
                jnp.bfloat16)
            rdma = pltpu.make_async_remote_copy(
                src_ref=stage_ref,
                dst_ref=rs_ref.at[st],
                send_sem=rs_send.at[st],
                recv_sem=rs_recv.at[st],
                device_id=(right,),
                device_id_type=pl.DeviceIdType.MESH,
            )
            rdma.start()
            rdma.wait()
            cr = lax.rem(d - 1 - st + 2 * N_DEV, N_DEV)
            acc_ref[pl.ds(cr * CHUNK, CHUNK), :] = (
                acc_ref[pl.ds(cr * CHUNK, CHUNK), :]
                + rs_ref[st].astype(jnp.float32))

        cm = lax.rem(d + 1, N_DEV)
        out_ref[pl.ds(cm * CHUNK, CHUNK), :] = (
            acc_ref[pl.ds(cm * CHUNK, CHUNK), :].astype(jnp.bfloat16))

        for st in range(N_DEV - 1):
            c = lax.rem(d + 1 - st + 2 * N_DEV, N_DEV)
            rdma = pltpu.make_async_remote_copy(
                src_ref=out_ref.at[pl.ds(c * CHUNK, CHUNK), :],
                dst_ref=out_ref.at[pl.ds(c * CHUNK, CHUNK), :],
                send_sem=ag_send.at[st],
                recv_sem=ag_recv.at[st],
                device_id=(right,),
                device_id_type=pl.DeviceIdType.MESH,
            )
            rdma.start()
            rdma.wait()

    out2 = pl.pallas_call(
        body,
        out_shape=jax.ShapeDtypeStruct((ROWS, D), jnp.bfloat16),
        in_specs=[pl.BlockSpec(memory_space=pltpu.VMEM)] * 5,
        out_specs=pl.BlockSpec(memory_space=pltpu.VMEM),
        scratch_shapes=[
            pltpu.VMEM((ROWS, D), jnp.float32),
            pltpu.VMEM((ROWS, D), jnp.bfloat16),
            pltpu.VMEM((N_DEV - 1, CHUNK, D), jnp.bfloat16),
            pltpu.VMEM((CHUNK, D), jnp.bfloat16),
            pltpu.SemaphoreType.DMA((N_DEV - 1,)),
            pltpu.SemaphoreType.DMA((N_DEV - 1,)),
            pltpu.SemaphoreType.DMA((N_DEV - 1,)),
            pltpu.SemaphoreType.DMA((N_DEV - 1,)),
        ],
    )(xb, wq, wk, wv, wo)
    return out2.reshape(B, SQ, D)


# device time: 73395 ns/iter; 2.3636x vs baseline; 1.7919x over previous
import jax
import jax.numpy as jnp
from jax import lax
from jax.experimental import pallas as pl
from jax.experimental.pallas import tpu as pltpu

N_DEV = 16
B, SQ, D = 4, 256, 1024
H_LOC, DH = 8, 128
ROWS = B * SQ
CHUNK = ROWS // N_DEV
SCALE = 0.08838834764831843


def kernel(x, Wq, Wo, Wk, Wv):
    xb = x.reshape(ROWS, D).astype(jnp.bfloat16)
    wq = Wq.astype(jnp.bfloat16)
    wk = Wk.astype(jnp.bfloat16)
    wv = Wv.astype(jnp.bfloat16)
    wo = Wo.astype(jnp.bfloat16)

    def body(x_ref, wq_ref, wk_ref, wv_ref, wo_ref, out_ref,
             acc_ref, attn_ref, stageA_ref, slotA_ref, stageB_ref, slotB_ref,
             sendA, recvA, sendB, recvB, sendC, recvC, sendD, recvD):
        d = lax.axis_index("i")
        w = lax.rem(d, 4)
        z = lax.div(d, 4)

        xv = x_ref[:]
        q = jnp.dot(xv, wq_ref[:],
                    preferred_element_type=jnp.float32).astype(jnp.bfloat16)
        k = jnp.dot(xv, wk_ref[:],
                    preferred_element_type=jnp.float32).astype(jnp.bfloat16)
        v = jnp.dot(xv, wv_ref[:],
                    preferred_element_type=jnp.float32).astype(jnp.bfloat16)

        for b in range(B):
            for h in range(H_LOC):
                qs = q[b * SQ:(b + 1) * SQ, h * DH:(h + 1) * DH]
                ks = k[b * SQ:(b + 1) * SQ, h * DH:(h + 1) * DH]
                vs = v[b * SQ:(b + 1) * SQ, h * DH:(h + 1) * DH]
                s = lax.dot_general(
                    qs, ks, (((1,), (1,)), ((), ())),
                    preferred_element_type=jnp.float32,
                ) * SCALE
                m = jnp.max(s, axis=1, keepdims=True)
                p = jnp.exp(s - m)
                l = jnp.sum(p, axis=1, keepdims=True)
                o = jnp.dot(p.astype(jnp.bfloat16), vs,
                            preferred_element_type=jnp.float32) / l
                attn_ref[b * SQ:(b + 1) * SQ, h * DH:(h + 1) * DH] = (
                    o.astype(jnp.bfloat16))

        acc_ref[:] = jnp.dot(attn_ref[:], wo_ref[:],
                             preferred_element_type=jnp.float32)

        QR, SC = 256, 64
        pending = []

        stageA_ref[:] = acc_ref[:].astype(jnp.bfloat16)
        for j in range(1, 4):
            wp = lax.rem(w + j, 4)
            peer = z * 4 + wp
            rdma = pltpu.make_async_remote_copy(
                src_ref=stageA_ref.at[pl.ds(wp * QR, QR), :],
                dst_ref=slotA_ref.at[pl.ds(w * QR, QR), :],
                send_sem=sendA.at[j],
                recv_sem=recvA.at[j],
                device_id=(peer,),
                device_id_type=pl.DeviceIdType.MESH,
            )
            rdma.start()
            pending.append(rdma)
        slotA_ref[pl.ds(w * QR, QR), :] = stageA_ref[pl.ds(w * QR, QR), :]
        for j in range(1, 4):
            ws = lax.rem(w - j + 4, 4)
            recv = pltpu.make_async_remote_copy(
                src_ref=stageA_ref.at[pl.ds(0, QR), :],
                dst_ref=slotA_ref.at[pl.ds(ws * QR, QR), :],
                send_sem=sendA.at[j],
                recv_sem=recvA.at[j],
                device_id=(d,),
                device_id_type=pl.DeviceIdType.MESH,
            )
            recv.wait_recv()
        qsum = (slotA_ref[pl.ds(0 * QR, QR), :].astype(jnp.float32)
                + slotA_ref[pl.ds(1 * QR, QR), :].astype(jnp.float32)
                + slotA_ref[pl.ds(2 * QR, QR), :].astype(jnp.float32)
                + slotA_ref[pl.ds(3 * QR, QR), :].astype(jnp.float32))

        stageB_ref[:] = qsum.astype(jnp.bfloat16)
        for j in range(1, 4):
            zp = lax.rem(z + j, 4)
            peer = zp * 4 + w
            rdma = pltpu.make_async_remote_copy(
                src_ref=stageB_ref.at[pl.ds(zp * SC, SC), :],
                dst_ref=slotB_ref.at[pl.ds(z * SC, SC), :],
                send_sem=sendB.at[j],
                recv_sem=recvB.at[j],
                device_id=(peer,),
                device_id_type=pl.DeviceIdType.MESH,
            )
            rdma.start()
            pending.append(rdma)
        slotB_ref[pl.ds(z * SC, SC), :] = stageB_ref[pl.ds(z * SC, SC), :]
        for j in range(1, 4):
            zs = lax.rem(z - j + 4, 4)
            recv = pltpu.make_async_remote_copy(
                src_ref=stageB_ref.at[pl.ds(0, SC), :],
                dst_ref=slotB_ref.at[pl.ds(zs * SC, SC), :],
                send_sem=sendB.at[j],
                recv_sem=recvB.at[j],
                device_id=(d,),
                device_id_type=pl.DeviceIdType.MESH,
            )
            recv.wait_recv()
        final = (slotB_ref[pl.ds(0 * SC, SC), :].astype(jnp.float32)
                 + slotB_ref[pl.ds(1 * SC, SC), :].astype(jnp.float32)
                 + slotB_ref[pl.ds(2 * SC, SC), :].astype(jnp.float32)
                 + slotB_ref[pl.ds(3 * SC, SC), :].astype(jnp.float32))
        my_rows = w * QR + z * SC
        out_ref[pl.ds(my_rows, SC), :] = final.astype(jnp.bfloat16)

        for j in range(1, 4):
            zp = lax.rem(z + j, 4)
            peer = zp * 4 + w
            rdma = pltpu.make_async_remote_copy(
                src_ref=out_ref.at[pl.ds(my_rows, SC), :],
                dst_ref=out_ref.at[pl.ds(my_rows, SC), :],
                send_sem=sendC.at[j],
                recv_sem=recvC.at[j],
                device_id=(peer,),
                device_id_type=pl.DeviceIdType.MESH,
            )
            rdma.start()
            pending.append(rdma)
        for j in range(1, 4):
            zs = lax.rem(z - j + 4, 4)
            recv = pltpu.make_async_remote_copy(
                src_ref=out_ref.at[pl.ds(0, SC), :],
                dst_ref=out_ref.at[pl.ds(w * QR + zs * SC, SC), :],
                send_sem=sendC.at[j],
                recv_sem=recvC.at[j],
                device_id=(d,),
                device_id_type=pl.DeviceIdType.MESH,
            )
            recv.wait_recv()

        for j in range(1, 4):
            wp = lax.rem(w + j, 4)
            peer = z * 4 + wp
            rdma = pltpu.make_async_remote_copy(
                src_ref=out_ref.at[pl.ds(w * QR, QR), :],
                dst_ref=out_ref.at[pl.ds(w * QR, QR), :],
                send_sem=sendD.at[j],
                recv_sem=recvD.at[j],
                device_id=(peer,),
                device_id_type=pl.DeviceIdType.MESH,
            )
            rdma.start()
            pending.append(rdma)
        for j in range(1, 4):
            ws = lax.rem(w - j + 4, 4)
            recv = pltpu.make_async_remote_copy(
                src_ref=out_ref.at[pl.ds(0, QR), :],
                dst_ref=out_ref.at[pl.ds(ws * QR, QR), :],
                send_sem=sendD.at[j],
                recv_sem=recvD.at[j],
                device_id=(d,),
                device_id_type=pl.DeviceIdType.MESH,
            )
            recv.wait_recv()

        for rdma in pending:
            rdma.wait_send()

    out2 = pl.pallas_call(
        body,
        out_shape=jax.ShapeDtypeStruct((ROWS, D), jnp.bfloat16),
        in_specs=[pl.BlockSpec(memory_space=pltpu.VMEM)] * 5,
        out_specs=pl.BlockSpec(memory_space=pltpu.VMEM),
        scratch_shapes=[
            pltpu.VMEM((ROWS, D), jnp.float32),
            pltpu.VMEM((ROWS, D), jnp.bfloat16),
            pltpu.VMEM((ROWS, D), jnp.bfloat16),
            pltpu.VMEM((ROWS, D), jnp.bfloat16),
            pltpu.VMEM((256, D), jnp.bfloat16),
            pltpu.VMEM((256, D), jnp.bfloat16),
            pltpu.SemaphoreType.DMA((4,)),
            pltpu.SemaphoreType.DMA((4,)),
            pltpu.SemaphoreType.DMA((4,)),
            pltpu.SemaphoreType.DMA((4,)),
            pltpu.SemaphoreType.DMA((4,)),
            pltpu.SemaphoreType.DMA((4,)),
            pltpu.SemaphoreType.DMA((4,)),
            pltpu.SemaphoreType.DMA((4,)),
        ],
    )(xb, wq, wk, wv, wo)
    return out2.reshape(B, SQ, D)


# device time: 67827 ns/iter; 2.5576x vs baseline; 1.0821x over previous
import jax
import jax.numpy as jnp
from jax import lax
from jax.experimental import pallas as pl
from jax.experimental.pallas import tpu as pltpu

N_DEV = 16
B, SQ, D = 4, 256, 1024
H_LOC, DH = 8, 128
ROWS = B * SQ
CHUNK = ROWS // N_DEV
SCALE = 0.08838834764831843


def kernel(x, Wq, Wo, Wk, Wv):
    xb = x.reshape(ROWS, D).astype(jnp.bfloat16)
    wq = Wq.astype(jnp.bfloat16)
    wk = Wk.astype(jnp.bfloat16)
    wv = Wv.astype(jnp.bfloat16)
    wo = Wo.astype(jnp.bfloat16)

    def body(x_ref, wq_ref, wk_ref, wv_ref, wo_ref, out_ref,
             attn_ref, stageA_ref, slotA_ref, stageB_ref, slotB_ref,
             sendA, recvA, sendB, recvB, sendC, recvC, sendD, recvD):
        d = lax.axis_index("i")
        w = lax.rem(d, 4)
        z = lax.div(d, 4)

        QR, SC = 256, 64
        pending = []

        for j in (1, 2, 3, 0):
            b = lax.rem(w + j, 4)
            r0 = b * QR
            xb_b = x_ref[pl.ds(r0, QR), :]
            qb = jnp.dot(xb_b, wq_ref[:],
                         preferred_element_type=jnp.float32).astype(
                             jnp.bfloat16)
            kb = jnp.dot(xb_b, wk_ref[:],
                         preferred_element_type=jnp.float32).astype(
                             jnp.bfloat16)
            vb = jnp.dot(xb_b, wv_ref[:],
                         preferred_element_type=jnp.float32).astype(
                             jnp.bfloat16)
            for h in range(H_LOC):
                qs = qb[:, h * DH:(h + 1) * DH]
                ks = kb[:, h * DH:(h + 1) * DH]
                vs = vb[:, h * DH:(h + 1) * DH]
                s = lax.dot_general(
                    qs, ks, (((1,), (1,)), ((), ())),
                    preferred_element_type=jnp.float32,
                ) * SCALE
                m = jnp.max(s, axis=1, keepdims=True)
                p = jnp.exp(s - m)
                l = jnp.sum(p, axis=1, keepdims=True)
                o = jnp.dot(p.astype(jnp.bfloat16), vs,
                            preferred_element_type=jnp.float32) / l
                attn_ref[:, h * DH:(h + 1) * DH] = o.astype(jnp.bfloat16)
            pb = jnp.dot(attn_ref[:], wo_ref[:],
                         preferred_element_type=jnp.float32).astype(
                             jnp.bfloat16)
            if j == 0:
                slotA_ref[pl.ds(w * QR, QR), :] = pb
            else:
                stageA_ref[pl.ds(r0, QR), :] = pb
                peer = z * 4 + b
                rdma = pltpu.make_async_remote_copy(
                    src_ref=stageA_ref.at[pl.ds(r0, QR), :],
                    dst_ref=slotA_ref.at[pl.ds(w * QR, QR), :],
                    send_sem=sendA.at[j],
                    recv_sem=recvA.at[j],
                    device_id=(peer,),
                    device_id_type=pl.DeviceIdType.MESH,
                )
                rdma.start()
                pending.append(rdma)

        for j in range(1, 4):
            ws = lax.rem(w - j + 4, 4)
            recv = pltpu.make_async_remote_copy(
                src_ref=stageA_ref.at[pl.ds(0, QR), :],
                dst_ref=slotA_ref.at[pl.ds(ws * QR, QR), :],
                send_sem=sendA.at[j],
                recv_sem=recvA.at[j],
                device_id=(d,),
                device_id_type=pl.DeviceIdType.MESH,
            )
            recv.wait_recv()
        qsum = (slotA_ref[pl.ds(0 * QR, QR), :].astype(jnp.float32)
                + slotA_ref[pl.ds(1 * QR, QR), :].astype(jnp.float32)
                + slotA_ref[pl.ds(2 * QR, QR), :].astype(jnp.float32)
                + slotA_ref[pl.ds(3 * QR, QR), :].astype(jnp.float32))

        stageB_ref[:] = qsum.astype(jnp.bfloat16)
        for j in range(1, 4):
            zp = lax.rem(z + j, 4)
            peer = zp * 4 + w
            rdma = pltpu.make_async_remote_copy(
                src_ref=stageB_ref.at[pl.ds(zp * SC, SC), :],
                dst_ref=slotB_ref.at[pl.ds(z * SC, SC), :],
                send_sem=sendB.at[j],
                recv_sem=recvB.at[j],
                device_id=(peer,),
                device_id_type=pl.DeviceIdType.MESH,
            )
            rdma.start()
            pending.append(rdma)
        slotB_ref[pl.ds(z * SC, SC), :] = stageB_ref[pl.ds(z * SC, SC), :]
        for j in range(1, 4):
            zs = lax.rem(z - j + 4, 4)
            recv = pltpu.make_async_remote_copy(
                src_ref=stageB_ref.at[pl.ds(0, SC), :],
                dst_ref=slotB_ref.at[pl.ds(zs * SC, SC), :],
                send_sem=sendB.at[j],
                recv_sem=recvB.at[j],
                device_id=(d,),
                device_id_type=pl.DeviceIdType.MESH,
            )
            recv.wait_recv()
        final = (slotB_ref[pl.ds(0 * SC, SC), :].astype(jnp.float32)
                 + slotB_ref[pl.ds(1 * SC, SC), :].astype(jnp.float32)
                 + slotB_ref[pl.ds(2 * SC, SC), :].astype(jnp.float32)
                 + slotB_ref[pl.ds(3 * SC, SC), :].astype(jnp.float32))
        my_rows = w * QR + z * SC
        out_ref[pl.ds(my_rows, SC), :] = final.astype(jnp.bfloat16)

        for j in range(1, 4):
            zp = lax.rem(z + j, 4)
            peer = zp * 4 + w
            rdma = pltpu.make_async_remote_copy(
                src_ref=out_ref.at[pl.ds(my_rows, SC), :],
                dst_ref=out_ref.at[pl.ds(my_rows, SC), :],
                send_sem=sendC.at[j],
                recv_sem=recvC.at[j],
                device_id=(peer,),
                device_id_type=pl.DeviceIdType.MESH,
            )
            rdma.start()
            pending.append(rdma)
        for j in range(1, 4):
            zs = lax.rem(z - j + 4, 4)
            recv = pltpu.make_async_remote_copy(
                src_ref=out_ref.at[pl.ds(0, SC), :],
                dst_ref=out_ref.at[pl.ds(w * QR + zs * SC, SC), :],
                send_sem=sendC.at[j],
                recv_sem=recvC.at[j],
                device_id=(d,),
                device_id_type=pl.DeviceIdType.MESH,
            )
            recv.wait_recv()

        for j in range(1, 4):
            wp = lax.rem(w + j, 4)
            peer = z * 4 + wp
            rdma = pltpu.make_async_remote_copy(
                src_ref=out_ref.at[pl.ds(w * QR, QR), :],
                dst_ref=out_ref.at[pl.ds(w * QR, QR), :],
                send_sem=sendD.at[j],
                recv_sem=recvD.at[j],
                device_id=(peer,),
                device_id_type=pl.DeviceIdType.MESH,
            )
            rdma.start()
            pending.append(rdma)
        for j in range(1, 4):
            ws = lax.rem(w - j + 4, 4)
            recv = pltpu.make_async_remote_copy(
                src_ref=out_ref.at[pl.ds(0, QR), :],
                dst_ref=out_ref.at[pl.ds(ws * QR, QR), :],
                send_sem=sendD.at[j],
                recv_sem=recvD.at[j],
                device_id=(d,),
                device_id_type=pl.DeviceIdType.MESH,
            )
            recv.wait_recv()

        for rdma in pending:
            rdma.wait_send()

    out2 = pl.pallas_call(
        body,
        out_shape=jax.ShapeDtypeStruct((ROWS, D), jnp.bfloat16),
        in_specs=[pl.BlockSpec(memory_space=pltpu.VMEM)] * 5,
        out_specs=pl.BlockSpec(memory_space=pltpu.VMEM),
        scratch_shapes=[
            pltpu.VMEM((256, D), jnp.bfloat16),
            pltpu.VMEM((ROWS, D), jnp.bfloat16),
            pltpu.VMEM((ROWS, D), jnp.bfloat16),
            pltpu.VMEM((256, D), jnp.bfloat16),
            pltpu.VMEM((256, D), jnp.bfloat16),
            pltpu.SemaphoreType.DMA((4,)),
            pltpu.SemaphoreType.DMA((4,)),
            pltpu.SemaphoreType.DMA((4,)),
            pltpu.SemaphoreType.DMA((4,)),
            pltpu.SemaphoreType.DMA((4,)),
            pltpu.SemaphoreType.DMA((4,)),
            pltpu.SemaphoreType.DMA((4,)),
            pltpu.SemaphoreType.DMA((4,)),
        ],
    )(xb, wq, wk, wv, wo)
    return out2.reshape(B, SQ, D)


# device time: 63950 ns/iter; 2.7127x vs baseline; 1.0606x over previous
import jax
import jax.numpy as jnp
from jax import lax
from jax.experimental import pallas as pl
from jax.experimental.pallas import tpu as pltpu

N_DEV = 16
B, SQ, D = 4, 256, 1024
H_LOC, DH = 8, 128
ROWS = B * SQ
CHUNK = ROWS // N_DEV
SCALE = 0.08838834764831843


def kernel(x, Wq, Wo, Wk, Wv):
    xb = x.reshape(ROWS, D).astype(jnp.bfloat16)
    wq = Wq.astype(jnp.bfloat16)
    wk = Wk.astype(jnp.bfloat16)
    wv = Wv.astype(jnp.bfloat16)
    wo = Wo.astype(jnp.bfloat16)

    def body(x_ref, wq_ref, wk_ref, wv_ref, wo_ref, out_ref,
             attn_ref, stageA_ref, slotA_ref, stageB_ref, slotB_ref,
             sendA, recvA, sendB, recvB, sendC, recvC, sendD, recvD):
        d = lax.axis_index("i")
        w = lax.rem(d, 4)
        z = lax.div(d, 4)

        QR, SC = 256, 64
        pending = []

        for j in (1, 2, 3, 0):
            b = lax.rem(w + j, 4)
            r0 = b * QR
            xb_b = x_ref[pl.ds(r0, QR), :]
            qb = jnp.dot(xb_b, wq_ref[:],
                         preferred_element_type=jnp.float32).astype(
                             jnp.bfloat16)
            kb = jnp.dot(xb_b, wk_ref[:],
                         preferred_element_type=jnp.float32).astype(
                             jnp.bfloat16)
            vb = jnp.dot(xb_b, wv_ref[:],
                         preferred_element_type=jnp.float32).astype(
                             jnp.bfloat16)
            for h in range(H_LOC):
                qs = qb[:, h * DH:(h + 1) * DH]
                ks = kb[:, h * DH:(h + 1) * DH]
                vs = vb[:, h * DH:(h + 1) * DH]
                s = lax.dot_general(
                    qs, ks, (((1,), (1,)), ((), ())),
                    preferred_element_type=jnp.float32,
                ) * SCALE
                m = jnp.max(s, axis=1, keepdims=True)
                p = jnp.exp(s - m)
                l = jnp.sum(p, axis=1, keepdims=True)
                o = jnp.dot(p.astype(jnp.bfloat16), vs,
                            preferred_element_type=jnp.float32) / l
                attn_ref[:, h * DH:(h + 1) * DH] = o.astype(jnp.bfloat16)
            pb = jnp.dot(attn_ref[:], wo_ref[:],
                         preferred_element_type=jnp.float32).astype(
                             jnp.bfloat16)
            if j == 0:
                slotA_ref[pl.ds(w * QR, QR), :] = pb
            else:
                stageA_ref[pl.ds(r0, QR), :] = pb
                peer = z * 4 + b
                rdma = pltpu.make_async_remote_copy(
                    src_ref=stageA_ref.at[pl.ds(r0, QR), :],
                    dst_ref=slotA_ref.at[pl.ds(w * QR, QR), :],
                    send_sem=sendA.at[j],
                    recv_sem=recvA.at[j],
                    device_id=(peer,),
                    device_id_type=pl.DeviceIdType.MESH,
                )
                rdma.start()
                pending.append(rdma)

        for j in range(1, 4):
            ws = lax.rem(w - j + 4, 4)
            recv = pltpu.make_async_remote_copy(
                src_ref=stageA_ref.at[pl.ds(0, QR), :],
                dst_ref=slotA_ref.at[pl.ds(ws * QR, QR), :],
                send_sem=sendA.at[j],
                recv_sem=recvA.at[j],
                device_id=(d,),
                device_id_type=pl.DeviceIdType.MESH,
            )
            recv.wait_recv()
        qsum = (slotA_ref[pl.ds(0 * QR, QR), :].astype(jnp.float32)
                + slotA_ref[pl.ds(1 * QR, QR), :].astype(jnp.float32)
                + slotA_ref[pl.ds(2 * QR, QR), :].astype(jnp.float32)
                + slotA_ref[pl.ds(3 * QR, QR), :].astype(jnp.float32))

        stageB_ref[:] = qsum.astype(jnp.bfloat16)
        for j in range(1, 4):
            zp = lax.rem(z + j, 4)
            peer = zp * 4 + w
            rdma = pltpu.make_async_remote_copy(
                src_ref=stageB_ref.at[pl.ds(zp * SC, SC), :],
                dst_ref=slotB_ref.at[pl.ds(z * SC, SC), :],
                send_sem=sendB.at[j],
                recv_sem=recvB.at[j],
                device_id=(peer,),
                device_id_type=pl.DeviceIdType.MESH,
            )
            rdma.start()
            pending.append(rdma)
        slotB_ref[pl.ds(z * SC, SC), :] = stageB_ref[pl.ds(z * SC, SC), :]
        for j in range(1, 4):
            zs = lax.rem(z - j + 4, 4)
            recv = pltpu.make_async_remote_copy(
                src_ref=stageB_ref.at[pl.ds(0, SC), :],
                dst_ref=slotB_ref.at[pl.ds(zs * SC, SC), :],
                send_sem=sendB.at[j],
                recv_sem=recvB.at[j],
                device_id=(d,),
                device_id_type=pl.DeviceIdType.MESH,
            )
            recv.wait_recv()
        final = (slotB_ref[pl.ds(0 * SC, SC), :].astype(jnp.float32)
                 + slotB_ref[pl.ds(1 * SC, SC), :].astype(jnp.float32)
                 + slotB_ref[pl.ds(2 * SC, SC), :].astype(jnp.float32)
                 + slotB_ref[pl.ds(3 * SC, SC), :].astype(jnp.float32))
        my_rows = w * QR + z * SC
        out_ref[pl.ds(my_rows, SC), :] = final.astype(jnp.bfloat16)

        for j in range(1, 4):
            zp = lax.rem(z + j, 4)
            peer = zp * 4 + w
            rdma = pltpu.make_async_remote_copy(
                src_ref=out_ref.at[pl.ds(my_rows, SC), :],
                dst_ref=out_ref.at[pl.ds(my_rows, SC), :],
                send_sem=sendC.at[j],
                recv_sem=recvC.at[j],
                device_id=(peer,),
                device_id_type=pl.DeviceIdType.MESH,
            )
            rdma.start()
            pending.append(rdma)

        for k in range(4):
            zs = lax.rem(z - k + 4, 4)
            rows_k = w * QR + zs * SC
            if k > 0:
                recv = pltpu.make_async_remote_copy(
                    src_ref=out_ref.at[pl.ds(0, SC), :],
                    dst_ref=out_ref.at[pl.ds(rows_k, SC), :],
                    send_sem=sendC.at[k],
                    recv_sem=recvC.at[k],
                    device_id=(d,),
                    device_id_type=pl.DeviceIdType.MESH,
                )
                recv.wait_recv()
            for j in range(1, 4):
                wp = lax.rem(w + j, 4)
                peer = z * 4 + wp
                rdma = pltpu.make_async_remote_copy(
                    src_ref=out_ref.at[pl.ds(rows_k, SC), :],
                    dst_ref=out_ref.at[pl.ds(rows_k, SC), :],
                    send_sem=sendD.at[k * 4 + j],
                    recv_sem=recvD.at[k * 4 + j],
                    device_id=(peer,),
                    device_id_type=pl.DeviceIdType.MESH,
                )
                rdma.start()
                pending.append(rdma)

        for k in range(4):
            zs = lax.rem(z - k + 4, 4)
            for j in range(1, 4):
                ws = lax.rem(w - j + 4, 4)
                recv = pltpu.make_async_remote_copy(
                    src_ref=out_ref.at[pl.ds(0, SC), :],
                    dst_ref=out_ref.at[pl.ds(ws * QR + zs * SC, SC), :],
                    send_sem=sendD.at[k * 4 + j],
                    recv_sem=recvD.at[k * 4 + j],
                    device_id=(d,),
                    device_id_type=pl.DeviceIdType.MESH,
                )
                recv.wait_recv()

        for rdma in pending:
            rdma.wait_send()

    out2 = pl.pallas_call(
        body,
        out_shape=jax.ShapeDtypeStruct((ROWS, D), jnp.bfloat16),
        in_specs=[pl.BlockSpec(memory_space=pltpu.VMEM)] * 5,
        out_specs=pl.BlockSpec(memory_space=pltpu.VMEM),
        scratch_shapes=[
            pltpu.VMEM((256, D), jnp.bfloat16),
            pltpu.VMEM((ROWS, D), jnp.bfloat16),
            pltpu.VMEM((ROWS, D), jnp.bfloat16),
            pltpu.VMEM((256, D), jnp.bfloat16),
            pltpu.VMEM((256, D), jnp.bfloat16),
            pltpu.SemaphoreType.DMA((4,)),
            pltpu.SemaphoreType.DMA((4,)),
            pltpu.SemaphoreType.DMA((4,)),
            pltpu.SemaphoreType.DMA((4,)),
            pltpu.SemaphoreType.DMA((4,)),
            pltpu.SemaphoreType.DMA((4,)),
            pltpu.SemaphoreType.DMA((16,)),
            pltpu.SemaphoreType.DMA((16,)),
        ],
    )(xb, wq, wk, wv, wo)
    return out2.reshape(B, SQ, D)
